# Initial kernel scaffold; baseline (speedup 1.0000x reference)
import jax
import jax.numpy as jnp
from jax.experimental import pallas as pl


def kernel(concept_tensor, times, W, b):
    raise NotImplementedError("write your pallas kernel here")


# speedup vs baseline: 72.9221x; 72.9221x over previous
import functools

import jax
import jax.numpy as jnp
from jax import lax
from jax.experimental import pallas as pl
from jax.experimental.pallas import tpu as pltpu
from jax.experimental.pallas import tpu_sc as plsc

FEATDIM = 8192
OUTDIM = 8
WINDOWS_DAYS = [7, 30, 90, 365]
PRED_DAY_UNIX = 1577836800
THRESHOLDS = [PRED_DAY_UNIX - d * 86400 for d in WINDOWS_DAYS]
NWIN = len(THRESHOLDS)
BSZ, SEQ, D2 = 256, 50, 16
EV = SEQ * D2
NGRP = EV // 16
TOTFEAT = NWIN * FEATDIM
NWORKERS = 32
PPW = BSZ // NWORKERS


def _sc_onehot(conc, times_b):
    mesh = plsc.VectorSubcoreMesh(core_axis_name="c", subcore_axis_name="s")

    @functools.partial(
        pl.kernel,
        mesh=mesh,
        compiler_params=pltpu.CompilerParams(needs_layout_passes=False),
        out_type=jax.ShapeDtypeStruct((BSZ, TOTFEAT), jnp.float32),
        scratch_types=[
            pltpu.VMEM((EV,), jnp.int32),
            pltpu.VMEM((EV,), jnp.int32),
            pltpu.VMEM((TOTFEAT,), jnp.float32),
            pltpu.VMEM((NWIN * EV,), jnp.int32),
        ],
    )
    def k(conc_hbm, times_hbm, oh_hbm, conc_v, time_v, oh_v, idx_v):
        wid = lax.axis_index("s") * 2 + lax.axis_index("c")
        ones = jnp.ones((16,), jnp.float32)
        zeros = jnp.zeros((16,), jnp.float32)

        def zero_all(j, carry):
            oh_v[pl.ds(j * 16, 16)] = zeros
            return carry

        lax.fori_loop(0, TOTFEAT // 16, zero_all, 0)

        for i in range(PPW):
            p = wid * PPW + i
            pltpu.sync_copy(conc_hbm.at[p], conc_v)
            pltpu.sync_copy(times_hbm.at[p], time_v)

            def scatter_grp(g, carry):
                c = conc_v[pl.ds(g * 16, 16)]
                t = time_v[pl.ds(g * 16, 16)]
                for wi in range(NWIN):
                    idx = jnp.where(t >= THRESHOLDS[wi], c, 0) + wi * FEATDIM
                    plsc.store_scatter(oh_v, [idx], ones)
                    idx_v[pl.ds(wi * EV + g * 16, 16)] = idx
                return carry

            lax.fori_loop(0, NGRP, scatter_grp, 0)
            pltpu.sync_copy(oh_v, oh_hbm.at[p])

            def zero_grp(j, carry):
                idx = idx_v[pl.ds(j * 16, 16)]
                plsc.store_scatter(oh_v, [idx], zeros)
                return carry

            lax.fori_loop(0, NWIN * NGRP, zero_grp, 0)

    return k(conc, times_b)


def _tc_matmul(oh, wt, bias):
    kblk = 2048
    nk = TOTFEAT // kblk

    def body(oh_ref, w_ref, b_ref, o_ref, acc_ref):
        kk = pl.program_id(0)

        @pl.when(kk == 0)
        def _():
            acc_ref[...] = jnp.zeros_like(acc_ref)

        acc_ref[...] += jnp.dot(
            oh_ref[...], w_ref[...], preferred_element_type=jnp.float32
        )

        @pl.when(kk == nk - 1)
        def _():
            o_ref[...] = acc_ref[...] + b_ref[...]

    return pl.pallas_call(
        body,
        grid=(nk,),
        in_specs=[
            pl.BlockSpec((BSZ, kblk), lambda k: (0, k)),
            pl.BlockSpec((kblk, OUTDIM), lambda k: (k, 0)),
            pl.BlockSpec((1, OUTDIM), lambda k: (0, 0)),
        ],
        out_specs=pl.BlockSpec((BSZ, OUTDIM), lambda k: (0, 0)),
        out_shape=jax.ShapeDtypeStruct((BSZ, OUTDIM), jnp.float32),
        scratch_shapes=[pltpu.VMEM((BSZ, OUTDIM), jnp.float32)],
    )(oh, wt, bias)


def kernel(concept_tensor, times, W, b):
    bsz = concept_tensor.shape[0]
    conc = concept_tensor.reshape(bsz, EV)
    times_b = jnp.broadcast_to(times[:, :, None], (bsz, SEQ, D2)).reshape(bsz, EV)
    oh = _sc_onehot(conc, times_b)
    return _tc_matmul(oh, W.T, b.reshape(1, OUTDIM))
